# speedup vs baseline: 1.8215x; 1.8215x over previous
import jax
import jax.numpy as jnp
from jax.experimental import pallas as pl
from jax.experimental.pallas import tpu as pltpu

_TOP_K = 8
_BT = 256


def _router_body(typ_ref, ctx_ref, w1a_ref, w1b_ref, b1_ref, w2_ref,
                 b2_ref, w3_ref, b3_ref, out_ref):
    typ = typ_ref[...].astype(jnp.bfloat16)
    ctx = ctx_ref[...].astype(jnp.bfloat16)
    h = jnp.dot(typ, w1a_ref[...], preferred_element_type=jnp.float32)
    h = h + jnp.dot(ctx, w1b_ref[...], preferred_element_type=jnp.float32)
    h = jnp.maximum(h + b1_ref[...], 0.0).astype(jnp.bfloat16)
    h = jnp.dot(h, w2_ref[...], preferred_element_type=jnp.float32)
    h = jnp.maximum(h + b2_ref[...], 0.0).astype(jnp.bfloat16)
    logits = jnp.dot(h, w3_ref[...], preferred_element_type=jnp.float32)
    logits = logits + b3_ref[...]

    bt, na = logits.shape
    col = jax.lax.broadcasted_iota(jnp.int32, (bt, na), 1)
    work = logits
    keep = jnp.zeros((bt, na), dtype=jnp.bool_)
    for _ in range(_TOP_K):
        m = jnp.max(work, axis=1, keepdims=True)
        cand = jnp.where(work == m, col, na)
        j = jnp.min(cand, axis=1, keepdims=True)
        sel = col == j
        keep = jnp.logical_or(keep, sel)
        work = jnp.where(sel, jnp.float32(-jnp.inf), work)
    m0 = jnp.max(logits, axis=1, keepdims=True)
    e = jnp.where(keep, jnp.exp(logits - m0), 0.0)
    out_ref[...] = e / jnp.sum(e, axis=1, keepdims=True)


def kernel(typology_embedding, context_features, W1, b1, W2, b2, W3, b3):
    tokens, typ_dim = typology_embedding.shape
    ctx_dim = context_features.shape[1]
    h1 = W1.shape[1]
    h2 = W2.shape[1]
    na = W3.shape[1]
    bt = min(_BT, tokens)
    grid = (tokens // bt,)

    w1a = W1[:typ_dim].astype(jnp.bfloat16)
    w1b = W1[typ_dim:].astype(jnp.bfloat16)
    w2 = W2.astype(jnp.bfloat16)
    w3 = W3.astype(jnp.bfloat16)
    b1r = b1.reshape(1, h1)
    b2r = b2.reshape(1, h2)
    b3r = b3.reshape(1, na)

    const = lambda i: (0, 0)
    return pl.pallas_call(
        _router_body,
        grid=grid,
        in_specs=[
            pl.BlockSpec((bt, typ_dim), lambda i: (i, 0)),
            pl.BlockSpec((bt, ctx_dim), lambda i: (i, 0)),
            pl.BlockSpec((typ_dim, h1), const),
            pl.BlockSpec((ctx_dim, h1), const),
            pl.BlockSpec((1, h1), const),
            pl.BlockSpec((h1, h2), const),
            pl.BlockSpec((1, h2), const),
            pl.BlockSpec((h2, na), const),
            pl.BlockSpec((1, na), const),
        ],
        out_specs=pl.BlockSpec((bt, na), lambda i: (i, 0)),
        out_shape=jax.ShapeDtypeStruct((tokens, na), jnp.float32),
    )(typology_embedding, context_features, w1a, w1b, b1r, w2, b2r, w3, b3r)


# speedup vs baseline: 2.0516x; 1.1263x over previous
import jax
import jax.numpy as jnp
from jax.experimental import pallas as pl
from jax.experimental.pallas import tpu as pltpu

_TOP_K = 8
_BT = 512


def _router_body(typ_ref, ctx_ref, w1a_ref, w1b_ref, b1_ref, w2_ref,
                 b2_ref, w3_ref, b3_ref, out_ref):
    typ = typ_ref[...].astype(jnp.bfloat16)
    ctx = ctx_ref[...].astype(jnp.bfloat16)
    h = jnp.dot(typ, w1a_ref[...], preferred_element_type=jnp.float32)
    h = h + jnp.dot(ctx, w1b_ref[...], preferred_element_type=jnp.float32)
    h = jnp.maximum(h + b1_ref[...], 0.0).astype(jnp.bfloat16)
    h = jnp.dot(h, w2_ref[...], preferred_element_type=jnp.float32)
    h = jnp.maximum(h + b2_ref[...], 0.0).astype(jnp.bfloat16)
    logits = jnp.dot(h, w3_ref[...], preferred_element_type=jnp.float32)
    logits = logits + b3_ref[...]

    bt, na = logits.shape
    col = jax.lax.broadcasted_iota(jnp.int32, (bt, na), 1)
    work = logits
    keep = jnp.zeros((bt, na), dtype=jnp.bool_)
    for _ in range(_TOP_K):
        m = jnp.max(work, axis=1, keepdims=True)
        cand = jnp.where(work == m, col, na)
        j = jnp.min(cand, axis=1, keepdims=True)
        sel = col == j
        keep = jnp.logical_or(keep, sel)
        work = jnp.where(sel, jnp.float32(-jnp.inf), work)
    m0 = jnp.max(logits, axis=1, keepdims=True)
    e = jnp.where(keep, jnp.exp(logits - m0), 0.0)
    out_ref[...] = e / jnp.sum(e, axis=1, keepdims=True)


def kernel(typology_embedding, context_features, W1, b1, W2, b2, W3, b3):
    tokens, typ_dim = typology_embedding.shape
    ctx_dim = context_features.shape[1]
    h1 = W1.shape[1]
    h2 = W2.shape[1]
    na = W3.shape[1]
    bt = min(_BT, tokens)
    grid = (tokens // bt,)

    w1a = W1[:typ_dim].astype(jnp.bfloat16)
    w1b = W1[typ_dim:].astype(jnp.bfloat16)
    w2 = W2.astype(jnp.bfloat16)
    w3 = W3.astype(jnp.bfloat16)
    b1r = b1.reshape(1, h1)
    b2r = b2.reshape(1, h2)
    b3r = b3.reshape(1, na)

    const = lambda i: (0, 0)
    return pl.pallas_call(
        _router_body,
        grid=grid,
        in_specs=[
            pl.BlockSpec((bt, typ_dim), lambda i: (i, 0)),
            pl.BlockSpec((bt, ctx_dim), lambda i: (i, 0)),
            pl.BlockSpec((typ_dim, h1), const),
            pl.BlockSpec((ctx_dim, h1), const),
            pl.BlockSpec((1, h1), const),
            pl.BlockSpec((h1, h2), const),
            pl.BlockSpec((1, h2), const),
            pl.BlockSpec((h2, na), const),
            pl.BlockSpec((1, na), const),
        ],
        out_specs=pl.BlockSpec((bt, na), lambda i: (i, 0)),
        out_shape=jax.ShapeDtypeStruct((tokens, na), jnp.float32),
    )(typology_embedding, context_features, w1a, w1b, b1r, w2, b2r, w3, b3r)


# speedup vs baseline: 2.1789x; 1.0620x over previous
import jax
import jax.numpy as jnp
from jax.experimental import pallas as pl
from jax.experimental.pallas import tpu as pltpu

_TOP_K = 8
_BT = 512


def _router_body(typ_ref, ctx_ref, w1a_ref, w1b_ref, b1_ref, w2_ref,
                 b2_ref, w3_ref, b3_ref, out_ref):
    typ = typ_ref[...].astype(jnp.bfloat16)
    ctx = ctx_ref[...].astype(jnp.bfloat16)
    h = jnp.dot(typ, w1a_ref[...], preferred_element_type=jnp.float32)
    h = h + jnp.dot(ctx, w1b_ref[...], preferred_element_type=jnp.float32)
    h = jnp.maximum(h + b1_ref[...], 0.0).astype(jnp.bfloat16)
    h = jnp.dot(h, w2_ref[...], preferred_element_type=jnp.float32)
    h = jnp.maximum(h + b2_ref[...], 0.0).astype(jnp.bfloat16)
    logits = jnp.dot(h, w3_ref[...], preferred_element_type=jnp.float32)
    logits = logits + b3_ref[...]

    bt, na = logits.shape
    colf = jax.lax.broadcasted_iota(jnp.int32, (bt, na), 1).astype(jnp.float32)
    work = logits
    keep = jnp.zeros((bt, na), dtype=jnp.bool_)
    m0 = None
    for t in range(_TOP_K):
        m = jnp.max(work, axis=1, keepdims=True)
        if t == 0:
            m0 = m
        cand = jnp.where(work == m, colf, jnp.float32(1e9))
        j = jnp.min(cand, axis=1, keepdims=True)
        sel = cand == j
        keep = jnp.logical_or(keep, sel)
        work = jnp.where(sel, jnp.float32(-jnp.inf), work)
    e = jnp.where(keep, jnp.exp(logits - m0), 0.0)
    out_ref[...] = e / jnp.sum(e, axis=1, keepdims=True)


def kernel(typology_embedding, context_features, W1, b1, W2, b2, W3, b3):
    tokens, typ_dim = typology_embedding.shape
    ctx_dim = context_features.shape[1]
    h1 = W1.shape[1]
    h2 = W2.shape[1]
    na = W3.shape[1]
    bt = min(_BT, tokens)
    grid = (tokens // bt,)

    w1a = W1[:typ_dim].astype(jnp.bfloat16)
    w1b = W1[typ_dim:].astype(jnp.bfloat16)
    w2 = W2.astype(jnp.bfloat16)
    w3 = W3.astype(jnp.bfloat16)
    b1r = b1.reshape(1, h1)
    b2r = b2.reshape(1, h2)
    b3r = b3.reshape(1, na)

    const = lambda i: (0, 0)
    return pl.pallas_call(
        _router_body,
        grid=grid,
        in_specs=[
            pl.BlockSpec((bt, typ_dim), lambda i: (i, 0)),
            pl.BlockSpec((bt, ctx_dim), lambda i: (i, 0)),
            pl.BlockSpec((typ_dim, h1), const),
            pl.BlockSpec((ctx_dim, h1), const),
            pl.BlockSpec((1, h1), const),
            pl.BlockSpec((h1, h2), const),
            pl.BlockSpec((1, h2), const),
            pl.BlockSpec((h2, na), const),
            pl.BlockSpec((1, na), const),
        ],
        out_specs=pl.BlockSpec((bt, na), lambda i: (i, 0)),
        out_shape=jax.ShapeDtypeStruct((tokens, na), jnp.float32),
    )(typology_embedding, context_features, w1a, w1b, b1r, w2, b2r, w3, b3r)


# speedup vs baseline: 2.2755x; 1.0444x over previous
import jax
import jax.numpy as jnp
from jax.experimental import pallas as pl
from jax.experimental.pallas import tpu as pltpu

_TOP_K = 8
_BT = 512


def _router_body(typ_ref, ctx_ref, w1a_ref, w1b_ref, b1_ref, w2_ref,
                 b2_ref, w3_ref, b3_ref, out_ref):
    typ = typ_ref[...].astype(jnp.bfloat16)
    ctx = ctx_ref[...].astype(jnp.bfloat16)
    h = jnp.dot(typ, w1a_ref[...], preferred_element_type=jnp.float32)
    h = h + jnp.dot(ctx, w1b_ref[...], preferred_element_type=jnp.float32)
    h = jnp.maximum(h + b1_ref[...], 0.0).astype(jnp.bfloat16)
    h = jnp.dot(h, w2_ref[...], preferred_element_type=jnp.float32)
    h = jnp.maximum(h + b2_ref[...], 0.0).astype(jnp.bfloat16)
    logits = jnp.dot(h, w3_ref[...], preferred_element_type=jnp.float32)
    logits = logits + b3_ref[...]

    bt, na = logits.shape
    col = jax.lax.broadcasted_iota(jnp.int32, (bt, na), 1)
    bits = jax.lax.bitcast_convert_type(logits, jnp.int32)
    keys = jax.lax.bitcast_convert_type(
        (bits & jnp.int32(-na)) | (jnp.int32(na - 1) - col), jnp.float32)
    work = keys
    keep = jnp.zeros((bt, na), dtype=jnp.bool_)
    m0 = None
    for t in range(_TOP_K):
        m = jnp.max(work, axis=1, keepdims=True)
        if t == 0:
            m0 = m
        sel = work == m
        keep = jnp.logical_or(keep, sel)
        work = jnp.where(sel, jnp.float32(-jnp.inf), work)
    e = jnp.where(keep, jnp.exp(logits - m0), 0.0)
    out_ref[...] = e / jnp.sum(e, axis=1, keepdims=True)


def kernel(typology_embedding, context_features, W1, b1, W2, b2, W3, b3):
    tokens, typ_dim = typology_embedding.shape
    ctx_dim = context_features.shape[1]
    h1 = W1.shape[1]
    h2 = W2.shape[1]
    na = W3.shape[1]
    bt = min(_BT, tokens)
    grid = (tokens // bt,)

    w1a = W1[:typ_dim].astype(jnp.bfloat16)
    w1b = W1[typ_dim:].astype(jnp.bfloat16)
    w2 = W2.astype(jnp.bfloat16)
    w3 = W3.astype(jnp.bfloat16)
    b1r = b1.reshape(1, h1)
    b2r = b2.reshape(1, h2)
    b3r = b3.reshape(1, na)

    const = lambda i: (0, 0)
    return pl.pallas_call(
        _router_body,
        grid=grid,
        in_specs=[
            pl.BlockSpec((bt, typ_dim), lambda i: (i, 0)),
            pl.BlockSpec((bt, ctx_dim), lambda i: (i, 0)),
            pl.BlockSpec((typ_dim, h1), const),
            pl.BlockSpec((ctx_dim, h1), const),
            pl.BlockSpec((1, h1), const),
            pl.BlockSpec((h1, h2), const),
            pl.BlockSpec((1, h2), const),
            pl.BlockSpec((h2, na), const),
            pl.BlockSpec((1, na), const),
        ],
        out_specs=pl.BlockSpec((bt, na), lambda i: (i, 0)),
        out_shape=jax.ShapeDtypeStruct((tokens, na), jnp.float32),
    )(typology_embedding, context_features, w1a, w1b, b1r, w2, b2r, w3, b3r)
